# Initial kernel scaffold; baseline (speedup 1.0000x reference)
import jax
import jax.numpy as jnp
from jax.experimental import pallas as pl


def kernel(x_user, x_item, edge_index_u2i, edge_index_i2u, W1l_u2i, b1l_u2i, W1r_u2i, W1l_i2u, b1l_i2u, W1r_i2u, W2l_u2i, b2l_u2i, W2r_u2i, W2l_i2u, b2l_i2u, W2r_i2u):
    raise NotImplementedError("write your pallas kernel here")


# speedup vs baseline: 6.6138x; 6.6138x over previous
import functools

import jax
import jax.numpy as jnp
from jax import lax
from jax.experimental import pallas as pl
from jax.experimental.pallas import tpu as pltpu
from jax.experimental.pallas import tpu_sc as plsc

N = 10000
D = 128
DA = 144
E = 320000

NC = 2
NS = 16
NW = NC * NS
C = 128
NCHUNK = 80
EPW = C * NCHUNK
EP = EPW * NW
NP = 10240
RPT = NP // NS

_mesh = plsc.VectorSubcoreMesh(core_axis_name="c", subcore_axis_name="s")


@functools.partial(
    pl.kernel,
    out_type=[jax.ShapeDtypeStruct((NC, NP, DA), jnp.float32)],
    mesh=_mesh,
    scratch_types=[
        pltpu.VMEM((NCHUNK, C), jnp.int32),
        pltpu.VMEM((NCHUNK, C), jnp.int32),
        pltpu.VMEM((C, DA), jnp.float32),
        pltpu.SemaphoreType.DMA,
        pltpu.VMEM_SHARED((NP, DA), jnp.float32),
    ],
    compiler_params=pltpu.CompilerParams(use_tc_tiling_on_sc=False))
def _agg(x_hbm, src_hbm, dst_hbm, out_sums, src_v, dst_v, rows_v, sem, acc_s):
    cid = lax.axis_index("c")
    sid = lax.axis_index("s")
    wid = cid * NS + sid

    pltpu.sync_copy(src_hbm.at[wid], src_v)
    pltpu.sync_copy(dst_hbm.at[wid], dst_v)

    z16 = jnp.zeros((16,), jnp.float32)

    def zrow(i, _):
        for j in range(DA // 16):
            rows_v[i, pl.ds(j * 16, 16)] = z16
        return 0
    lax.fori_loop(0, C, zrow, 0)

    for k in range(RPT // C):
        pltpu.sync_copy(rows_v, acc_s.at[pl.ds(sid * RPT + k * C, C)])

    plsc.subcore_barrier()

    def step(i, _):
        pltpu.async_copy(x_hbm.at[src_v.at[i]], rows_v, sem).wait()
        pltpu.sync_copy(rows_v, acc_s.at[dst_v.at[i]], add=True)
        return 0
    lax.fori_loop(0, NCHUNK, step, 0)

    plsc.subcore_barrier()

    pltpu.sync_copy(acc_s.at[pl.ds(sid * RPT, RPT)],
                    out_sums.at[cid, pl.ds(sid * RPT, RPT)])


BN = 1000


def _make_dense(relu: bool, aug_out: bool):
    def body(p_ref, x_ref, wl_ref, b_ref, wr_ref, o_ref):
        s = p_ref[0, :, :D] + p_ref[1, :, :D]
        cnt = jnp.maximum(p_ref[0, :, D:D + 1] + p_ref[1, :, D:D + 1], 1.0)
        agg = s / cnt
        y = jnp.dot(agg, wl_ref[...], preferred_element_type=jnp.float32)
        y = y + jnp.dot(x_ref[:, :D], wr_ref[...],
                        preferred_element_type=jnp.float32)
        y = y + b_ref[...]
        if relu:
            y = jnp.maximum(y, 0.0)
        if aug_out:
            y = jnp.concatenate(
                [y, jnp.ones((BN, 1), jnp.float32),
                 jnp.zeros((BN, DA - D - 1), jnp.float32)], axis=1)
        o_ref[...] = y

    return pl.pallas_call(
        body,
        grid=(N // BN,),
        in_specs=[
            pl.BlockSpec((NC, BN, DA), lambda i: (0, i, 0)),
            pl.BlockSpec((BN, DA), lambda i: (i, 0)),
            pl.BlockSpec((D, D), lambda i: (0, 0)),
            pl.BlockSpec((1, D), lambda i: (0, 0)),
            pl.BlockSpec((D, D), lambda i: (0, 0)),
        ],
        out_specs=pl.BlockSpec((BN, DA if aug_out else D), lambda i: (i, 0)),
        out_shape=jax.ShapeDtypeStruct((N, DA if aug_out else D),
                                       jnp.float32),
    )


_dense_aug = _make_dense(True, True)
_dense_out = _make_dense(False, False)


def _pad_edges(edge_index):
    src, dst = edge_index[0], edge_index[1]
    pad = EP - E
    ar = jnp.arange(pad, dtype=jnp.int32)
    pad_src = (ar * 37) % N
    pad_dst = N + ar % (NP - N)
    src_p = jnp.concatenate([src, pad_src]).reshape(NW, NCHUNK, C)
    dst_p = jnp.concatenate([dst, pad_dst]).reshape(NW, NCHUNK, C)
    return src_p, dst_p


def _augment(x):
    return jnp.concatenate(
        [x, jnp.ones((N, 1), jnp.float32),
         jnp.zeros((N, DA - D - 1), jnp.float32)], axis=1)


def kernel(x_user, x_item, edge_index_u2i, edge_index_i2u,
           W1l_u2i, b1l_u2i, W1r_u2i, W1l_i2u, b1l_i2u, W1r_i2u,
           W2l_u2i, b2l_u2i, W2r_u2i, W2l_i2u, b2l_i2u, W2r_i2u):
    src_u2i, dst_u2i = _pad_edges(edge_index_u2i)
    src_i2u, dst_i2u = _pad_edges(edge_index_i2u)
    xu = _augment(x_user)
    xi = _augment(x_item)
    b1l_u2i = b1l_u2i.reshape(1, D)
    b1l_i2u = b1l_i2u.reshape(1, D)
    b2l_u2i = b2l_u2i.reshape(1, D)
    b2l_i2u = b2l_i2u.reshape(1, D)

    (sums1_i,) = _agg(xu, src_u2i, dst_u2i)
    (sums1_u,) = _agg(xi, src_i2u, dst_i2u)
    h_item = _dense_aug(sums1_i, xi, W1l_u2i, b1l_u2i, W1r_u2i)
    h_user = _dense_aug(sums1_u, xu, W1l_i2u, b1l_i2u, W1r_i2u)
    (sums2_i,) = _agg(h_user, src_u2i, dst_u2i)
    (sums2_u,) = _agg(h_item, src_i2u, dst_i2u)
    o_item = _dense_out(sums2_i, h_item, W2l_u2i, b2l_u2i, W2r_u2i)
    o_user = _dense_out(sums2_u, h_user, W2l_i2u, b2l_i2u, W2r_i2u)
    return (o_user, o_item)


# speedup vs baseline: 9.6749x; 1.4628x over previous
import functools

import jax
import jax.numpy as jnp
from jax import lax
from jax.experimental import pallas as pl
from jax.experimental.pallas import tpu as pltpu
from jax.experimental.pallas import tpu_sc as plsc

N = 10000
D = 128
DA = 144
E = 320000

NC = 2
NS = 16
NW = NC * NS
C = 128
NCHUNK = 80
EPW = C * NCHUNK
EP = EPW * NW
NP = 10112
RPT = NP // NS
IB = 4
NB = NCHUNK // IB
NPAIR = NB // 2

_mesh = plsc.VectorSubcoreMesh(core_axis_name="c", subcore_axis_name="s")


@functools.partial(
    pl.kernel,
    out_type=[jax.ShapeDtypeStruct((NC, NP, DA), jnp.float32)],
    mesh=_mesh,
    scratch_types=[
        pltpu.VMEM((IB, C), jnp.int32),
        pltpu.VMEM((IB, C), jnp.int32),
        pltpu.VMEM((IB, C), jnp.int32),
        pltpu.VMEM((IB, C), jnp.int32),
        pltpu.VMEM((C, DA), jnp.float32),
        pltpu.VMEM((C, DA), jnp.float32),
        pltpu.SemaphoreType.DMA,
        pltpu.SemaphoreType.DMA,
        pltpu.SemaphoreType.DMA,
        pltpu.SemaphoreType.DMA,
        pltpu.VMEM_SHARED((NP, DA), jnp.float32),
    ],
    compiler_params=pltpu.CompilerParams(use_tc_tiling_on_sc=False))
def _agg(x_hbm, src_hbm, dst_hbm, out_sums,
         src_i0, src_i1, dst_i0, dst_i1, rows0, rows1,
         gsem0, gsem1, isem0, isem1, acc_s):
    cid = lax.axis_index("c")
    sid = lax.axis_index("s")
    wid = cid * NS + sid
    src_i = (src_i0, src_i1)
    dst_i = (dst_i0, dst_i1)
    rows = (rows0, rows1)
    gsem = (gsem0, gsem1)
    isem = (isem0, isem1)

    z16 = jnp.zeros((16,), jnp.float32)

    def zrow(i, _):
        for j in range(DA // 16):
            rows0[i, pl.ds(j * 16, 16)] = z16
        return 0
    lax.fori_loop(0, C, zrow, 0)

    for k in range(RPT // C):
        pltpu.sync_copy(rows0, acc_s.at[pl.ds(sid * RPT + k * C, C)])
    pltpu.sync_copy(rows0.at[pl.ds(0, RPT - (RPT // C) * C)],
                    acc_s.at[pl.ds(sid * RPT + (RPT // C) * C,
                                   RPT - (RPT // C) * C)])

    plsc.subcore_barrier()

    def fetch_idx(batch, par, sem):
        pltpu.async_copy(src_hbm.at[wid, pl.ds(batch * IB, IB)],
                         src_i[par], sem)
        pltpu.async_copy(dst_hbm.at[wid, pl.ds(batch * IB, IB)],
                         dst_i[par], sem)

    def wait_idx(par, sem):
        pltpu.make_async_copy(src_hbm.at[wid, pl.ds(0, IB)],
                              src_i[par], sem).wait()
        pltpu.make_async_copy(dst_hbm.at[wid, pl.ds(0, IB)],
                              dst_i[par], sem).wait()

    def fire(i, jj):
        par = (jj // IB) % 2
        buf = jj % 2
        pltpu.async_copy(x_hbm.at[src_i[par].at[jj % IB]], rows[buf],
                         gsem[buf])

    def consume(i, jj):
        par = (jj // IB) % 2
        buf = jj % 2
        pltpu.make_async_copy(x_hbm.at[src_i[par].at[jj % IB]], rows[buf],
                              gsem[buf]).wait()
        pltpu.sync_copy(rows[buf], acc_s.at[dst_i[par].at[jj % IB]],
                        add=True)

    fetch_idx(0, 0, isem0)
    wait_idx(0, isem0)
    fire(0, 0)
    fire(1, 1)

    def pair(mm, last):
        base = mm * 8
        fetch_idx(2 * mm + 1, 1, isem1)
        for jj in range(8):
            i = base + jj
            if jj == 2:
                wait_idx(1, isem1)
            if jj == 4 and not last:
                fetch_idx(2 * mm + 2, 0, isem0)
            if jj == 6 and not last:
                wait_idx(0, isem0)
            consume(i, jj)
            if jj + 2 < 8:
                fire(i + 2, jj + 2)
            elif not last:
                fire(i + 2, jj - 6)

    def pair_loop(mm, _):
        pair(mm, False)
        return 0
    lax.fori_loop(0, NPAIR - 1, pair_loop, 0)
    pair(NPAIR - 1, True)

    plsc.subcore_barrier()

    pltpu.sync_copy(acc_s.at[pl.ds(sid * RPT, RPT)],
                    out_sums.at[cid, pl.ds(sid * RPT, RPT)])


BN = 1000


def _make_dense(relu: bool, aug_out: bool):
    def body(p_ref, x_ref, wl_ref, b_ref, wr_ref, o_ref):
        s = p_ref[0, :, :D] + p_ref[1, :, :D]
        cnt = jnp.maximum(p_ref[0, :, D:D + 1] + p_ref[1, :, D:D + 1], 1.0)
        agg = s / cnt
        y = jnp.dot(agg, wl_ref[...], preferred_element_type=jnp.float32)
        y = y + jnp.dot(x_ref[:, :D], wr_ref[...],
                        preferred_element_type=jnp.float32)
        y = y + b_ref[...]
        if relu:
            y = jnp.maximum(y, 0.0)
        if aug_out:
            y = jnp.concatenate(
                [y, jnp.ones((BN, 1), jnp.float32),
                 jnp.zeros((BN, DA - D - 1), jnp.float32)], axis=1)
        o_ref[...] = y

    return pl.pallas_call(
        body,
        grid=(N // BN,),
        in_specs=[
            pl.BlockSpec((NC, BN, DA), lambda i: (0, i, 0)),
            pl.BlockSpec((BN, DA), lambda i: (i, 0)),
            pl.BlockSpec((D, D), lambda i: (0, 0)),
            pl.BlockSpec((1, D), lambda i: (0, 0)),
            pl.BlockSpec((D, D), lambda i: (0, 0)),
        ],
        out_specs=pl.BlockSpec((BN, DA if aug_out else D), lambda i: (i, 0)),
        out_shape=jax.ShapeDtypeStruct((N, DA if aug_out else D),
                                       jnp.float32),
    )


_dense_aug = _make_dense(True, True)
_dense_out = _make_dense(False, False)


def _pad_edges(edge_index):
    src, dst = edge_index[0], edge_index[1]
    pad = EP - E
    ar = jnp.arange(pad, dtype=jnp.int32)
    pad_src = (ar * 37) % N
    pad_dst = N + ar % (NP - N)
    src_p = jnp.concatenate([src, pad_src]).reshape(NW, NCHUNK, C)
    dst_p = jnp.concatenate([dst, pad_dst]).reshape(NW, NCHUNK, C)
    return src_p, dst_p


def _augment(x):
    return jnp.concatenate(
        [x, jnp.ones((N, 1), jnp.float32),
         jnp.zeros((N, DA - D - 1), jnp.float32)], axis=1)


def kernel(x_user, x_item, edge_index_u2i, edge_index_i2u,
           W1l_u2i, b1l_u2i, W1r_u2i, W1l_i2u, b1l_i2u, W1r_i2u,
           W2l_u2i, b2l_u2i, W2r_u2i, W2l_i2u, b2l_i2u, W2r_i2u):
    src_u2i, dst_u2i = _pad_edges(edge_index_u2i)
    src_i2u, dst_i2u = _pad_edges(edge_index_i2u)
    xu = _augment(x_user)
    xi = _augment(x_item)
    b1l_u2i = b1l_u2i.reshape(1, D)
    b1l_i2u = b1l_i2u.reshape(1, D)
    b2l_u2i = b2l_u2i.reshape(1, D)
    b2l_i2u = b2l_i2u.reshape(1, D)

    (sums1_i,) = _agg(xu, src_u2i, dst_u2i)
    (sums1_u,) = _agg(xi, src_i2u, dst_i2u)
    h_item = _dense_aug(sums1_i, xi, W1l_u2i, b1l_u2i, W1r_u2i)
    h_user = _dense_aug(sums1_u, xu, W1l_i2u, b1l_i2u, W1r_i2u)
    (sums2_i,) = _agg(h_user, src_u2i, dst_u2i)
    (sums2_u,) = _agg(h_item, src_i2u, dst_i2u)
    o_item = _dense_out(sums2_i, h_item, W2l_u2i, b2l_u2i, W2r_u2i)
    o_user = _dense_out(sums2_u, h_user, W2l_i2u, b2l_i2u, W2r_i2u)
    return (o_user, o_item)
